# Initial kernel scaffold; baseline (speedup 1.0000x reference)
import jax
import jax.numpy as jnp
from jax.experimental import pallas as pl


def kernel(x):
    raise NotImplementedError("write your pallas kernel here")


# speedup vs baseline: 755.3893x; 755.3893x over previous
import jax
import jax.numpy as jnp
from jax.experimental import pallas as pl
from jax.experimental.pallas import tpu as pltpu

_ROWS_PER_BLOCK = 256


def _lwta_body(x_ref, o_ref):
    x = x_ref[...]
    left = pltpu.roll(x, x.shape[1] - 1, axis=1)
    right = pltpu.roll(x, 1, axis=1)
    lane = jax.lax.broadcasted_iota(jnp.int32, x.shape, dimension=1)
    even = (lane & 1) == 0
    neighbor = jnp.where(even, left, right)
    win = (even & (x >= neighbor)) | (~even & (x > neighbor))
    o_ref[...] = jnp.where(win, x, jnp.zeros_like(x))


def kernel(x):
    orig_shape = x.shape
    n_last = orig_shape[-1]
    x2 = x.reshape(-1, n_last)
    rows = x2.shape[0]
    block = _ROWS_PER_BLOCK
    grid = rows // block
    out = pl.pallas_call(
        _lwta_body,
        grid=(grid,),
        in_specs=[pl.BlockSpec((block, n_last), lambda i: (i, 0))],
        out_specs=pl.BlockSpec((block, n_last), lambda i: (i, 0)),
        out_shape=jax.ShapeDtypeStruct((rows, n_last), x.dtype),
    )(x2)
    return out.reshape(orig_shape)


# speedup vs baseline: 796.7098x; 1.0547x over previous
import jax
import jax.numpy as jnp
from jax.experimental import pallas as pl
from jax.experimental.pallas import tpu as pltpu

_ROWS_PER_BLOCK = 256
_LANE = 128


def _lwta_body(x_ref, o_ref):
    n = x_ref.shape[1]
    shape = (x_ref.shape[0], _LANE)
    lane = jax.lax.broadcasted_iota(jnp.int32, shape, dimension=1)
    even = (lane & 1) == 0
    odd = ~even
    zero = jnp.zeros(shape, x_ref.dtype)
    for j in range(n // _LANE):
        sl = pl.ds(j * _LANE, _LANE)
        x = x_ref[:, sl]
        left = pltpu.roll(x, _LANE - 1, axis=1)
        right = pltpu.roll(x, 1, axis=1)
        win = (even & (x >= left)) | (odd & (x > right))
        o_ref[:, sl] = jnp.where(win, x, zero)


def kernel(x):
    orig_shape = x.shape
    n_last = orig_shape[-1]
    x2 = x.reshape(-1, n_last)
    rows = x2.shape[0]
    block = _ROWS_PER_BLOCK
    grid = (rows // block,)
    out = pl.pallas_call(
        _lwta_body,
        grid=grid,
        in_specs=[pl.BlockSpec((block, n_last), lambda i: (i, 0))],
        out_specs=pl.BlockSpec((block, n_last), lambda i: (i, 0)),
        out_shape=jax.ShapeDtypeStruct((rows, n_last), x.dtype),
    )(x2)
    return out.reshape(orig_shape)
